# baseline (speedup 1.0000x reference)
import jax
import jax.numpy as jnp
from jax.experimental import pallas as pl
from jax.experimental.pallas import tpu as pltpu

_TARGET_RATE = 0.5
_INT_MIN = -2147483648


def _mask_kernel(imp_ref, mask_ref, *, k):
    x = imp_ref[...]
    bits = jax.lax.bitcast_convert_type(x, jnp.int32)
    key = jnp.where(bits >= 0, bits, jnp.int32(_INT_MIN) - bits)
    n = key.shape[-1]
    idx = jax.lax.broadcasted_iota(jnp.int32, key.shape, 2)

    def count_ge(v):
        return jnp.sum((key >= v).astype(jnp.int32))

    base = jnp.where(count_ge(jnp.int32(0)) >= k, jnp.int32(0),
                     jnp.int32(_INT_MIN))

    def body(i, b):
        cand = b | (jnp.int32(1) << (jnp.int32(30) - i))
        return jnp.where(count_ge(cand) >= k, cand, b)

    v = jax.lax.fori_loop(0, 31, body, base)

    cnt_gt = jnp.sum((key > v).astype(jnp.int32))
    r = jnp.int32(k) - cnt_gt
    tie = key == v

    def body2(i, carry):
        lo, hi = carry
        mid = (lo + hi) // 2
        g = jnp.sum((tie & (idx <= mid)).astype(jnp.int32))
        ok = g >= r
        return jnp.where(ok, lo, mid + 1), jnp.where(ok, mid, hi)

    _, cut = jax.lax.fori_loop(
        0, 16, body2, (jnp.int32(0), jnp.int32(n - 1)))

    mask = (key > v) | (tie & (idx <= cut))
    mask_ref[...] = mask.astype(jnp.float32)


def _mul_kernel(s_ref, m_ref, o_ref):
    o_ref[...] = s_ref[...] * m_ref[...]


def _build_mask(imp, k):
    B, _, N = imp.shape
    import functools
    return pl.pallas_call(
        functools.partial(_mask_kernel, k=k),
        grid=(B,),
        in_specs=[pl.BlockSpec((1, 1, N), lambda b: (b, 0, 0))],
        out_specs=pl.BlockSpec((1, 1, N), lambda b: (b, 0, 0)),
        out_shape=jax.ShapeDtypeStruct((B, 1, N), jnp.float32),
    )(imp)


def kernel(spikes, importance, training):
    T, B, C, H, W = spikes.shape
    N = H * W
    k = max(1, int(_TARGET_RATE * N))

    imp = importance.reshape(B, 1, N)
    mask = _build_mask(imp, k)

    s = spikes.reshape(T * B, C, N)
    cb = next(c for c in range(min(16, C), 0, -1) if C % c == 0)
    out = pl.pallas_call(
        _mul_kernel,
        grid=(T * B, C // cb),
        in_specs=[
            pl.BlockSpec((1, cb, N), lambda i, j: (i, j, 0)),
            pl.BlockSpec((1, 1, N), lambda i, j: (i % B, 0, 0)),
        ],
        out_specs=pl.BlockSpec((1, cb, N), lambda i, j: (i, j, 0)),
        out_shape=jax.ShapeDtypeStruct((T * B, C, N), jnp.float32),
        compiler_params=pltpu.CompilerParams(
            dimension_semantics=("parallel", "parallel")),
    )(s, mask)
    return out.reshape(T, B, C, H, W)


# speedup vs baseline: 1.1000x; 1.1000x over previous
import functools

import jax
import jax.numpy as jnp
from jax import lax
from jax.experimental import pallas as pl
from jax.experimental.pallas import tpu as pltpu
from jax.experimental.pallas import tpu_sc as plsc

_TARGET_RATE = 0.5
_INT_MIN = -2147483648


def _build_mask_sc(imp_flat, B, N, k):
    info = plsc.get_sparse_core_info()
    NC, NS = info.num_cores, info.num_subcores
    WPB = (NC * NS) // B
    CH = N // WPB
    NV = CH // 16
    ROW = 16
    mesh = plsc.VectorSubcoreMesh(core_axis_name="c", subcore_axis_name="s")

    @functools.partial(
        pl.kernel,
        mesh=mesh,
        compiler_params=pltpu.CompilerParams(needs_layout_passes=False),
        out_type=jax.ShapeDtypeStruct((B * N,), jnp.float32),
        scratch_types=[
            pltpu.VMEM((CH,), jnp.float32),
            pltpu.VMEM((CH,), jnp.int32),
            pltpu.VMEM((CH,), jnp.float32),
            pltpu.VMEM((ROW,), jnp.int32),
            pltpu.VMEM((WPB * ROW,), jnp.int32),
            pltpu.VMEM_SHARED((2 * NS * ROW,), jnp.int32),
        ],
    )
    def sc_mask(imp_hbm, out_hbm, x_v, key_v, out_v, stage_v, team_v, counts_sm):
        c = lax.axis_index("c")
        s = lax.axis_index("s")
        batch = c * (B // NC) + s // WPB
        slot = s % WPB
        team_lo = (s // WPB) * WPB
        base = batch * N + slot * CH

        pltpu.sync_copy(imp_hbm.at[pl.ds(base, CH)], x_v)

        def keys_body(i, carry):
            xv = x_v[pl.ds(i * 16, 16)]
            bits = lax.bitcast_convert_type(xv, jnp.int32)
            key_v[pl.ds(i * 16, 16)] = jnp.where(
                bits >= 0, bits, jnp.int32(_INT_MIN) - bits)
            return carry

        lax.fori_loop(0, NV, keys_body, jnp.int32(0))

        def local_count_ge(thr):
            def body(i, acc):
                kv = key_v[pl.ds(i * 16, 16)]
                return acc + jnp.where(kv >= thr, jnp.int32(1), jnp.int32(0))
            acc = lax.fori_loop(0, NV, body, jnp.zeros((16,), jnp.int32))
            return jnp.sum(acc)

        def local_count_gt(thr):
            def body(i, acc):
                kv = key_v[pl.ds(i * 16, 16)]
                return acc + jnp.where(kv > thr, jnp.int32(1), jnp.int32(0))
            acc = lax.fori_loop(0, NV, body, jnp.zeros((16,), jnp.int32))
            return jnp.sum(acc)

        def share_round(parity, val):
            stage_v[...] = jnp.full((ROW,), val, jnp.int32)
            off = parity * (NS * ROW)
            pltpu.sync_copy(stage_v, counts_sm.at[pl.ds(off + s * ROW, ROW)])
            plsc.subcore_barrier()
            pltpu.sync_copy(counts_sm.at[pl.ds(off + team_lo * ROW, WPB * ROW)],
                            team_v)

        def team_total():
            def body(r, acc):
                return acc + team_v[pl.ds(r * ROW, ROW)]
            acc = lax.fori_loop(0, WPB, body, jnp.zeros((16,), jnp.int32))
            return jnp.max(acc)

        def radix_body(i, basev):
            cand = basev + (jnp.int32(1) << (jnp.int32(31) - i))
            cnt = local_count_ge(cand)
            share_round(i % 2, cnt)
            total = team_total()
            return jnp.where(total >= k, cand, basev)

        v = lax.fori_loop(0, 32, radix_body, jnp.int32(_INT_MIN))

        cnt_gt = local_count_gt(v)
        share_round(0, cnt_gt)
        r_need = jnp.int32(k) - team_total()

        tie_local = local_count_ge(v) - cnt_gt
        share_round(1, tie_local)

        def prefix_body(rr, acc):
            rowmax = jnp.max(team_v[pl.ds(rr * ROW, ROW)])
            return acc + jnp.where(rr < slot, rowmax, jnp.int32(0))

        tie_before = lax.fori_loop(0, WPB, prefix_body, jnp.int32(0))
        q = jnp.minimum(jnp.maximum(r_need - tie_before, jnp.int32(0)),
                        tie_local)

        def mask_body(i, run):
            kv = key_v[pl.ds(i * 16, 16)]
            gt = kv > v
            tie = kv == v
            t01 = jnp.where(tie, jnp.int32(1), jnp.int32(0))
            csum = lax.cumsum(t01)
            accept = tie & ((run + csum) <= q)
            out_v[pl.ds(i * 16, 16)] = jnp.where(
                gt | accept, jnp.float32(1.0), jnp.float32(0.0))
            return run + jnp.max(csum)

        lax.fori_loop(0, NV, mask_body, jnp.int32(0))
        pltpu.sync_copy(out_v, out_hbm.at[pl.ds(base, CH)])

    return sc_mask(imp_flat)


def _mul_kernel(s_ref, m_ref, o_ref):
    o_ref[...] = s_ref[...] * m_ref[...]


def kernel(spikes, importance, training):
    T, B, C, H, W = spikes.shape
    N = H * W
    k = max(1, int(_TARGET_RATE * N))

    mask = _build_mask_sc(importance.reshape(B * N), B, N, k)
    mask = mask.reshape(B, 1, N)

    s = spikes.reshape(T * B, C, N)
    cb = next(c for c in range(min(16, C), 0, -1) if C % c == 0)
    out = pl.pallas_call(
        _mul_kernel,
        grid=(T * B, C // cb),
        in_specs=[
            pl.BlockSpec((1, cb, N), lambda i, j: (i, j, 0)),
            pl.BlockSpec((1, 1, N), lambda i, j: (i % B, 0, 0)),
        ],
        out_specs=pl.BlockSpec((1, cb, N), lambda i, j: (i, j, 0)),
        out_shape=jax.ShapeDtypeStruct((T * B, C, N), jnp.float32),
        compiler_params=pltpu.CompilerParams(
            dimension_semantics=("parallel", "parallel")),
    )(s, mask)
    return out.reshape(T, B, C, H, W)


# speedup vs baseline: 1.1047x; 1.0042x over previous
import functools

import jax
import jax.numpy as jnp
from jax import lax
from jax.experimental import pallas as pl
from jax.experimental.pallas import tpu as pltpu
from jax.experimental.pallas import tpu_sc as plsc

_TARGET_RATE = 0.5
_INT_MIN = -2147483648


def _build_mask_sc(imp_flat, B, N, k):
    info = plsc.get_sparse_core_info()
    NC, NS = info.num_cores, info.num_subcores
    WPB = (NC * NS) // B
    CH = N // WPB
    NV = CH // 16
    ROW = 16
    mesh = plsc.VectorSubcoreMesh(core_axis_name="c", subcore_axis_name="s")

    @functools.partial(
        pl.kernel,
        mesh=mesh,
        compiler_params=pltpu.CompilerParams(needs_layout_passes=False),
        out_type=jax.ShapeDtypeStruct((B * N,), jnp.float32),
        scratch_types=[
            pltpu.VMEM((CH,), jnp.float32),
            pltpu.VMEM((CH,), jnp.int32),
            pltpu.VMEM((CH,), jnp.float32),
            pltpu.VMEM((ROW,), jnp.int32),
            pltpu.VMEM((WPB * ROW,), jnp.int32),
            pltpu.VMEM_SHARED((2 * NS * ROW,), jnp.int32),
        ],
    )
    def sc_mask(imp_hbm, out_hbm, x_v, key_v, out_v, stage_v, team_v, counts_sm):
        c = lax.axis_index("c")
        s = lax.axis_index("s")
        batch = c * (B // NC) + s // WPB
        slot = s % WPB
        team_lo = (s // WPB) * WPB
        base = batch * N + slot * CH

        pltpu.sync_copy(imp_hbm.at[pl.ds(base, CH)], x_v)

        def keys_body(i, carry):
            xv = x_v[pl.ds(i * 16, 16)]
            bits = lax.bitcast_convert_type(xv, jnp.int32)
            key_v[pl.ds(i * 16, 16)] = jnp.where(
                bits >= 0, bits, jnp.int32(_INT_MIN) - bits)
            return carry

        lax.fori_loop(0, NV, keys_body, jnp.int32(0))

        def local_count_ge(thr):
            def body(i, acc):
                kv = key_v[pl.ds(i * 16, 16)]
                return acc + jnp.where(kv >= thr, jnp.int32(1), jnp.int32(0))
            acc = lax.fori_loop(0, NV, body, jnp.zeros((16,), jnp.int32))
            return jnp.sum(acc)

        def local_count_gt(thr):
            def body(i, acc):
                kv = key_v[pl.ds(i * 16, 16)]
                return acc + jnp.where(kv > thr, jnp.int32(1), jnp.int32(0))
            acc = lax.fori_loop(0, NV, body, jnp.zeros((16,), jnp.int32))
            return jnp.sum(acc)

        def share_round(parity, val):
            stage_v[...] = jnp.full((ROW,), val, jnp.int32)
            off = parity * (NS * ROW)
            pltpu.sync_copy(stage_v, counts_sm.at[pl.ds(off + s * ROW, ROW)])
            plsc.subcore_barrier()
            pltpu.sync_copy(counts_sm.at[pl.ds(off + team_lo * ROW, WPB * ROW)],
                            team_v)

        def team_total():
            def body(r, acc):
                return acc + team_v[pl.ds(r * ROW, ROW)]
            acc = lax.fori_loop(0, WPB, body, jnp.zeros((16,), jnp.int32))
            return jnp.max(acc)

        def radix_body(i, basev):
            cand = basev + (jnp.int32(1) << (jnp.int32(31) - i))
            cnt = local_count_ge(cand)
            share_round(i % 2, cnt)
            total = team_total()
            return jnp.where(total >= k, cand, basev)

        v = lax.fori_loop(0, 32, radix_body, jnp.int32(_INT_MIN))

        cnt_gt = local_count_gt(v)
        share_round(0, cnt_gt)
        r_need = jnp.int32(k) - team_total()

        tie_local = local_count_ge(v) - cnt_gt
        share_round(1, tie_local)

        def prefix_body(rr, acc):
            rowmax = jnp.max(team_v[pl.ds(rr * ROW, ROW)])
            return acc + jnp.where(rr < slot, rowmax, jnp.int32(0))

        tie_before = lax.fori_loop(0, WPB, prefix_body, jnp.int32(0))
        q = jnp.minimum(jnp.maximum(r_need - tie_before, jnp.int32(0)),
                        tie_local)

        def mask_body(i, run):
            kv = key_v[pl.ds(i * 16, 16)]
            gt = kv > v
            tie = kv == v
            t01 = jnp.where(tie, jnp.int32(1), jnp.int32(0))
            csum = lax.cumsum(t01)
            accept = tie & ((run + csum) <= q)
            out_v[pl.ds(i * 16, 16)] = jnp.where(
                gt | accept, jnp.float32(1.0), jnp.float32(0.0))
            return run + jnp.max(csum)

        lax.fori_loop(0, NV, mask_body, jnp.int32(0))
        pltpu.sync_copy(out_v, out_hbm.at[pl.ds(base, CH)])

    return sc_mask(imp_flat)


def _mul_kernel(s_ref, m_ref, o_ref):
    o_ref[...] = s_ref[...] * m_ref[...]


def kernel(spikes, importance, training):
    T, B, C, H, W = spikes.shape
    N = H * W
    k = max(1, int(_TARGET_RATE * N))

    mask = _build_mask_sc(importance.reshape(B * N), B, N, k)
    mask = mask.reshape(B, 1, N)

    s = spikes.reshape(T * B, C, N)
    cb = next(c for c in range(min(32, C), 0, -1) if C % c == 0)
    out = pl.pallas_call(
        _mul_kernel,
        grid=(T * B, C // cb),
        in_specs=[
            pl.BlockSpec((1, cb, N), lambda i, j: (i, j, 0)),
            pl.BlockSpec((1, 1, N), lambda i, j: (i % B, 0, 0)),
        ],
        out_specs=pl.BlockSpec((1, cb, N), lambda i, j: (i, j, 0)),
        out_shape=jax.ShapeDtypeStruct((T * B, C, N), jnp.float32),
        compiler_params=pltpu.CompilerParams(
            dimension_semantics=("parallel", "parallel")),
    )(s, mask)
    return out.reshape(T, B, C, H, W)


# speedup vs baseline: 1.1052x; 1.0004x over previous
import functools

import jax
import jax.numpy as jnp
from jax import lax
from jax.experimental import pallas as pl
from jax.experimental.pallas import tpu as pltpu
from jax.experimental.pallas import tpu_sc as plsc

_TARGET_RATE = 0.5
_INT_MIN = -2147483648


def _build_mask_sc(imp_flat, B, N, k):
    info = plsc.get_sparse_core_info()
    NC, NS = info.num_cores, info.num_subcores
    WPB = (NC * NS) // B
    CH = N // WPB
    NV = CH // 16
    U = 8 if NV % 8 == 0 else 1
    NG = NV // U
    ROW = 16
    RPW = 3
    mesh = plsc.VectorSubcoreMesh(core_axis_name="c", subcore_axis_name="s")

    @functools.partial(
        pl.kernel,
        mesh=mesh,
        compiler_params=pltpu.CompilerParams(needs_layout_passes=False),
        out_type=jax.ShapeDtypeStruct((B * N,), jnp.float32),
        scratch_types=[
            pltpu.VMEM((CH,), jnp.float32),
            pltpu.VMEM((CH,), jnp.int32),
            pltpu.VMEM((CH,), jnp.float32),
            pltpu.VMEM((RPW * ROW,), jnp.int32),
            pltpu.VMEM((WPB * RPW * ROW,), jnp.int32),
            pltpu.VMEM_SHARED((2 * NS * RPW * ROW,), jnp.int32),
        ],
    )
    def sc_mask(imp_hbm, out_hbm, x_v, key_v, out_v, stage_v, team_v, counts_sm):
        c = lax.axis_index("c")
        s = lax.axis_index("s")
        batch = c * (B // NC) + s // WPB
        slot = s % WPB
        team_lo = (s // WPB) * WPB
        base = batch * N + slot * CH

        pltpu.sync_copy(imp_hbm.at[pl.ds(base, CH)], x_v)

        def keys_body(g, carry):
            for u in range(U):
                i = g * U + u
                bits = lax.bitcast_convert_type(x_v[pl.ds(i * 16, 16)],
                                                jnp.int32)
                key_v[pl.ds(i * 16, 16)] = jnp.where(
                    bits >= 0, bits, jnp.int32(_INT_MIN) - bits)
            return carry

        lax.fori_loop(0, NG, keys_body, jnp.int32(0))

        one = jnp.int32(1)
        zero16 = jnp.zeros((16,), jnp.int32)

        def publish(parity, vecs):
            for j, vec in enumerate(vecs):
                stage_v[pl.ds(j * ROW, ROW)] = vec
            off = (parity * NS + s) * (RPW * ROW)
            pltpu.sync_copy(stage_v, counts_sm.at[pl.ds(off, RPW * ROW)])
            plsc.subcore_barrier()
            toff = (parity * NS + team_lo) * (RPW * ROW)
            pltpu.sync_copy(counts_sm.at[pl.ds(toff, WPB * RPW * ROW)], team_v)

        def team_sum(j):
            def body(r, acc):
                return acc + team_v[pl.ds(r * (RPW * ROW) + j * ROW, ROW)]
            return jnp.sum(lax.fori_loop(0, WPB, body, zero16))

        def radix_body(t, basev):
            shift = jnp.int32(30) - 2 * t
            cand1 = basev + (one << shift)
            cand2 = basev + (jnp.int32(2) << shift)
            cand3 = basev + (jnp.int32(3) << shift)

            def scan(g, accs):
                a1, a2, a3 = accs
                for u in range(U):
                    kv = key_v[pl.ds((g * U + u) * 16, 16)]
                    a1 = a1 + jnp.where(kv >= cand1, one, 0)
                    a2 = a2 + jnp.where(kv >= cand2, one, 0)
                    a3 = a3 + jnp.where(kv >= cand3, one, 0)
                return a1, a2, a3

            a1, a2, a3 = lax.fori_loop(0, NG, scan, (zero16, zero16, zero16))
            publish(t % 2, [a1, a2, a3])
            t1, t2, t3 = team_sum(0), team_sum(1), team_sum(2)
            return jnp.where(
                t3 >= k, cand3,
                jnp.where(t2 >= k, cand2, jnp.where(t1 >= k, cand1, basev)))

        v = lax.fori_loop(0, 16, radix_body, jnp.int32(_INT_MIN))

        def count_scan(g, accs):
            ag, at_ = accs
            for u in range(U):
                kv = key_v[pl.ds((g * U + u) * 16, 16)]
                ag = ag + jnp.where(kv > v, one, 0)
                at_ = at_ + jnp.where(kv == v, one, 0)
            return ag, at_

        accg, acct = lax.fori_loop(0, NG, count_scan, (zero16, zero16))
        publish(0, [accg, acct])
        r_need = jnp.int32(k) - team_sum(0)
        tie_local = jnp.sum(acct)

        def prefix_body(rr, acc):
            rowsum = jnp.sum(team_v[pl.ds(rr * (RPW * ROW) + ROW, ROW)])
            return acc + jnp.where(rr < slot, rowsum, jnp.int32(0))

        tie_before = lax.fori_loop(0, WPB, prefix_body, jnp.int32(0))
        q = jnp.minimum(jnp.maximum(r_need - tie_before, jnp.int32(0)),
                        tie_local)

        fone, fzero = jnp.float32(1.0), jnp.float32(0.0)

        def write_plain(_):
            def body(g, carry):
                for u in range(U):
                    i = g * U + u
                    kv = key_v[pl.ds(i * 16, 16)]
                    out_v[pl.ds(i * 16, 16)] = jnp.where(kv > v, fone, fzero)
                return carry
            return lax.fori_loop(0, NG, body, jnp.int32(0))

        def write_all_ties(_):
            def body(g, carry):
                for u in range(U):
                    i = g * U + u
                    kv = key_v[pl.ds(i * 16, 16)]
                    out_v[pl.ds(i * 16, 16)] = jnp.where(kv >= v, fone, fzero)
                return carry
            return lax.fori_loop(0, NG, body, jnp.int32(0))

        def write_cumsum(_):
            def body(i, run):
                kv = key_v[pl.ds(i * 16, 16)]
                tie = kv == v
                csum = lax.cumsum(jnp.where(tie, one, 0))
                accept = tie & ((run + csum) <= q)
                out_v[pl.ds(i * 16, 16)] = jnp.where(
                    (kv > v) | accept, fone, fzero)
                return run + jnp.max(csum)
            return lax.fori_loop(0, NV, body, jnp.int32(0))

        _ = lax.cond(
            q == 0, write_plain,
            lambda _: lax.cond(q == tie_local, write_all_ties,
                               write_cumsum, 0),
            0)

        pltpu.sync_copy(out_v, out_hbm.at[pl.ds(base, CH)])

    return sc_mask(imp_flat)


def _mul_kernel(s_ref, m_ref, o_ref):
    o_ref[...] = s_ref[...] * m_ref[...]


def kernel(spikes, importance, training):
    T, B, C, H, W = spikes.shape
    N = H * W
    k = max(1, int(_TARGET_RATE * N))

    mask = _build_mask_sc(importance.reshape(B * N), B, N, k)
    mask = mask.reshape(B, 1, N)

    s = spikes.reshape(T * B, C, N)
    cb = next(c for c in range(min(32, C), 0, -1) if C % c == 0)
    out = pl.pallas_call(
        _mul_kernel,
        grid=(T * B, C // cb),
        in_specs=[
            pl.BlockSpec((1, cb, N), lambda i, j: (i, j, 0)),
            pl.BlockSpec((1, 1, N), lambda i, j: (i % B, 0, 0)),
        ],
        out_specs=pl.BlockSpec((1, cb, N), lambda i, j: (i, j, 0)),
        out_shape=jax.ShapeDtypeStruct((T * B, C, N), jnp.float32),
        compiler_params=pltpu.CompilerParams(
            dimension_semantics=("parallel", "parallel")),
    )(s, mask)
    return out.reshape(T, B, C, H, W)
